# baseline (speedup 1.0000x reference)
import functools

import jax
import jax.numpy as jnp
from jax import lax
from jax.experimental import pallas as pl
from jax.experimental.pallas import tpu as pltpu
from jax.experimental.pallas import tpu_sc as plsc

Q = 262144
N = 128
NC = 2
NS = 16
L = 16
NW = NC * NS
PER_W = Q // NW
BLK = 2048
NBLK = PER_W // BLK
GROUPS = BLK // L

_mesh = plsc.VectorSubcoreMesh(core_axis_name="c", subcore_axis_name="s")


@functools.partial(
    pl.kernel,
    mesh=_mesh,
    out_type=[
        jax.ShapeDtypeStruct((Q * 8,), jnp.int32),
        jax.ShapeDtypeStruct((Q * 8,), jnp.float32),
    ],
    scratch_types=[
        pltpu.VMEM((BLK * 3,), jnp.float32),
        pltpu.VMEM((L,), jnp.float32),
        pltpu.VMEM((BLK * 8,), jnp.int32),
        pltpu.VMEM((BLK * 8,), jnp.int32),
        pltpu.VMEM((BLK * 8,), jnp.float32),
        pltpu.SemaphoreType.DMA,
    ],
    compiler_params=pltpu.CompilerParams(needs_layout_passes=False),
)
def _sc_lookup(pts, svec, grid, out_ids, out_w, pts_v, sv_v, idx_v, ids_v, w_v, sem):
    wid = lax.axis_index("s") * NC + lax.axis_index("c")
    iota = lax.iota(jnp.int32, L)
    lanes8 = iota * 8
    iota3 = iota * 3
    zero16 = jnp.zeros((L,), jnp.int32)

    pltpu.sync_copy(svec, sv_v)
    sx = plsc.load_gather(sv_v, [zero16])
    sy = plsc.load_gather(sv_v, [zero16 + 1])
    sz = plsc.load_gather(sv_v, [zero16 + 2])
    ox = plsc.load_gather(sv_v, [zero16 + 3])
    oy = plsc.load_gather(sv_v, [zero16 + 4])
    oz = plsc.load_gather(sv_v, [zero16 + 5])

    for b in range(NBLK):
        p0 = wid * PER_W + b * BLK
        pltpu.sync_copy(pts.at[pl.ds(p0 * 3, BLK * 3)], pts_v)

        def body(j, carry):
            gbase = iota3 + j * (L * 3)

            def axis(col, sc, off):
                u = plsc.load_gather(pts_v, [gbase + col])
                u = u * sc + off
                u = jnp.minimum(jnp.maximum(u, 0.0), 127.0)
                i = jnp.minimum(u.astype(jnp.int32), 126)
                f = u - i.astype(jnp.float32)
                return i, f

            xi, fx = axis(0, sx, ox)
            yi, fy = axis(1, sy, oy)
            zi, fz = axis(2, sz, oz)

            base = zi * (N * N) + yi * N + xi
            gx = 1.0 - fx
            gy = 1.0 - fy
            gz = 1.0 - fz
            w00 = gz * gy
            w01 = gz * fy
            w10 = fz * gy
            w11 = fz * fy
            weights = (w00 * gx, w00 * fx, w01 * gx, w01 * fx,
                       w10 * gx, w10 * fx, w11 * gx, w11 * fx)
            offs = (0, 1, N, N + 1,
                    N * N, N * N + 1, N * N + N, N * N + N + 1)

            pos = lanes8 + j * (L * 8)
            for c in range(8):
                plsc.store_scatter(idx_v, [pos + c], base + offs[c])
                plsc.store_scatter(w_v, [pos + c], weights[c])
            return carry

        lax.fori_loop(0, GROUPS, body, 0)

        pltpu.async_copy(grid.at[idx_v], ids_v, sem).wait()
        pltpu.sync_copy(ids_v, out_ids.at[pl.ds(p0 * 8, BLK * 8)])
        pltpu.sync_copy(w_v, out_w.at[pl.ds(p0 * 8, BLK * 8)])


def kernel(points, grid_id, bbox_min, bbox_max):
    scale = (N - 1.0) / (bbox_max - bbox_min)
    offset = -bbox_min * scale
    svec = jnp.concatenate([scale, offset, jnp.zeros((10,), jnp.float32)])
    ids_f, w_f = _sc_lookup(points.reshape(-1), svec, grid_id.reshape(-1))
    return ids_f.reshape(Q, 8), w_f.reshape(Q, 8)


# speedup vs baseline: 3.8665x; 3.8665x over previous
import functools

import jax
import jax.numpy as jnp
from jax import lax
from jax.experimental import pallas as pl
from jax.experimental.pallas import tpu as pltpu
from jax.experimental.pallas import tpu_sc as plsc

Q = 262144
N = 128
NC = 2
NS = 16
L = 16
NW = NC * NS
PER_W = Q // NW
BLK = 2048
NBLK = PER_W // BLK
GROUPS = BLK // L

_mesh = plsc.VectorSubcoreMesh(core_axis_name="c", subcore_axis_name="s")


@functools.partial(
    pl.kernel,
    mesh=_mesh,
    out_type=[
        jax.ShapeDtypeStruct((Q * 8,), jnp.int32),
        jax.ShapeDtypeStruct((Q * 8,), jnp.float32),
    ],
    scratch_types=[
        pltpu.VMEM((BLK,), jnp.float32),
        pltpu.VMEM((BLK,), jnp.float32),
        pltpu.VMEM((BLK,), jnp.float32),
        pltpu.VMEM((BLK * 8,), jnp.int32),
        pltpu.VMEM((BLK * 8,), jnp.int32),
        pltpu.VMEM((BLK * 8,), jnp.float32),
        pltpu.SemaphoreType.DMA,
    ],
    compiler_params=pltpu.CompilerParams(needs_layout_passes=False),
)
def _sc_lookup(ux, uy, uz, grid, out_ids, out_w, ux_v, uy_v, uz_v, idx_v, ids_v, w_v, sem):
    wid = lax.axis_index("s") * NC + lax.axis_index("c")

    for b in range(NBLK):
        p0 = wid * PER_W + b * BLK
        pltpu.sync_copy(ux.at[pl.ds(p0, BLK)], ux_v)
        pltpu.sync_copy(uy.at[pl.ds(p0, BLK)], uy_v)
        pltpu.sync_copy(uz.at[pl.ds(p0, BLK)], uz_v)

        def body(j, carry):
            s = j * L

            def axis(ref):
                u = ref[pl.ds(s, L)]
                u = jnp.minimum(jnp.maximum(u, 0.0), 127.0)
                i = jnp.minimum(u.astype(jnp.int32), 126)
                f = u - i.astype(jnp.float32)
                return i, f

            xi, fx = axis(ux_v)
            yi, fy = axis(uy_v)
            zi, fz = axis(uz_v)

            base = zi * (N * N) + yi * N + xi
            gx = 1.0 - fx
            gy = 1.0 - fy
            gz = 1.0 - fz
            w00 = gz * gy
            w01 = gz * fy
            w10 = fz * gy
            w11 = fz * fy
            weights = (w00 * gx, w00 * fx, w01 * gx, w01 * fx,
                       w10 * gx, w10 * fx, w11 * gx, w11 * fx)
            offs = (0, 1, N, N + 1,
                    N * N, N * N + 1, N * N + N, N * N + N + 1)

            pos0 = (j >> 3) * (8 * 128) + (j & 7) * L
            for c in range(8):
                idx_v[pl.ds(pos0 + c * 128, L)] = base + offs[c]
                w_v[pl.ds(pos0 + c * 128, L)] = weights[c]
            return carry

        lax.fori_loop(0, GROUPS, body, 0)

        pltpu.async_copy(grid.at[idx_v], ids_v, sem).wait()
        pltpu.sync_copy(ids_v, out_ids.at[pl.ds(p0 * 8, BLK * 8)])
        pltpu.sync_copy(w_v, out_w.at[pl.ds(p0 * 8, BLK * 8)])


def kernel(points, grid_id, bbox_min, bbox_max):
    scale = (N - 1.0) / (bbox_max - bbox_min)
    ut = ((points - bbox_min) * scale).T
    ids_f, w_f = _sc_lookup(ut[0], ut[1], ut[2], grid_id.reshape(-1))
    ids = ids_f.reshape(Q // 128, 8, 128).swapaxes(1, 2).reshape(Q, 8)
    w = w_f.reshape(Q // 128, 8, 128).swapaxes(1, 2).reshape(Q, 8)
    return ids, w


# speedup vs baseline: 4.3088x; 1.1144x over previous
import functools

import jax
import jax.numpy as jnp
from jax import lax
from jax.experimental import pallas as pl
from jax.experimental.pallas import tpu as pltpu
from jax.experimental.pallas import tpu_sc as plsc

Q = 262144
N = 128
NC = 2
NS = 16
L = 16
NW = NC * NS
PER_W = Q // NW
BLK = 2048
NBLK = PER_W // BLK
GROUPS = BLK // L

_mesh = plsc.VectorSubcoreMesh(core_axis_name="c", subcore_axis_name="s")


@functools.partial(
    pl.kernel,
    mesh=_mesh,
    out_type=[
        jax.ShapeDtypeStruct((Q * 8,), jnp.int32),
        jax.ShapeDtypeStruct((Q * 8,), jnp.float32),
    ],
    scratch_types=[
        pltpu.VMEM((BLK,), jnp.float32),
        pltpu.VMEM((BLK,), jnp.float32),
        pltpu.VMEM((BLK,), jnp.float32),
        pltpu.VMEM((BLK * 8,), jnp.int32),
        pltpu.VMEM((BLK * 8,), jnp.int32),
        pltpu.VMEM((BLK * 8,), jnp.float32),
        pltpu.VMEM((BLK * 8,), jnp.int32),
        pltpu.VMEM((BLK * 8,), jnp.int32),
        pltpu.VMEM((BLK * 8,), jnp.float32),
        pltpu.SemaphoreType.DMA,
        pltpu.SemaphoreType.DMA,
    ],
    compiler_params=pltpu.CompilerParams(needs_layout_passes=False),
)
def _sc_lookup(ux, uy, uz, grid, out_ids, out_w, ux_v, uy_v, uz_v,
               idx0_v, ids0_v, w0_v, idx1_v, ids1_v, w1_v, sem0, sem1):
    wid = lax.axis_index("s") * NC + lax.axis_index("c")
    bufs = ((idx0_v, ids0_v, w0_v, sem0), (idx1_v, ids1_v, w1_v, sem1))
    pending = [None, None]

    for b in range(NBLK):
        idx_v, ids_v, w_v, sem = bufs[b % 2]
        p0 = wid * PER_W + b * BLK
        pltpu.sync_copy(ux.at[pl.ds(p0, BLK)], ux_v)
        pltpu.sync_copy(uy.at[pl.ds(p0, BLK)], uy_v)
        pltpu.sync_copy(uz.at[pl.ds(p0, BLK)], uz_v)

        def body(j, carry):
            s = j * L

            def axis(ref):
                u = ref[pl.ds(s, L)]
                u = jnp.minimum(jnp.maximum(u, 0.0), 127.0)
                i = jnp.minimum(u.astype(jnp.int32), 126)
                f = u - i.astype(jnp.float32)
                return i, f

            xi, fx = axis(ux_v)
            yi, fy = axis(uy_v)
            zi, fz = axis(uz_v)

            base = zi * (N * N) + yi * N + xi
            gx = 1.0 - fx
            gy = 1.0 - fy
            gz = 1.0 - fz
            w00 = gz * gy
            w01 = gz * fy
            w10 = fz * gy
            w11 = fz * fy
            weights = (w00 * gx, w00 * fx, w01 * gx, w01 * fx,
                       w10 * gx, w10 * fx, w11 * gx, w11 * fx)
            offs = (0, 1, N, N + 1,
                    N * N, N * N + 1, N * N + N, N * N + N + 1)

            pos0 = (j >> 3) * (8 * 128) + (j & 7) * L
            for c in range(8):
                idx_v[pl.ds(pos0 + c * 128, L)] = base + offs[c]
                w_v[pl.ds(pos0 + c * 128, L)] = weights[c]
            return carry

        lax.fori_loop(0, GROUPS, body, 0)

        prev = pending[1 - b % 2]
        gather = pltpu.async_copy(grid.at[idx_v], ids_v, sem)
        if prev is not None:
            prev_cp, prev_p0, prev_ids = prev
            prev_cp.wait()
            pltpu.sync_copy(prev_ids, out_ids.at[pl.ds(prev_p0 * 8, BLK * 8)])
        pending[b % 2] = (gather, p0, ids_v)
        pltpu.sync_copy(w_v, out_w.at[pl.ds(p0 * 8, BLK * 8)])

    last_cp, last_p0, last_ids = pending[(NBLK - 1) % 2]
    last_cp.wait()
    pltpu.sync_copy(last_ids, out_ids.at[pl.ds(last_p0 * 8, BLK * 8)])


def kernel(points, grid_id, bbox_min, bbox_max):
    scale = (N - 1.0) / (bbox_max - bbox_min)
    ut = ((points - bbox_min) * scale).T
    ids_f, w_f = _sc_lookup(ut[0], ut[1], ut[2], grid_id.reshape(-1))
    ids = ids_f.reshape(Q // 128, 8, 128).swapaxes(1, 2).reshape(Q, 8)
    w = w_f.reshape(Q // 128, 8, 128).swapaxes(1, 2).reshape(Q, 8)
    return ids, w
